# Initial kernel scaffold; baseline (speedup 1.0000x reference)
import jax
import jax.numpy as jnp
from jax.experimental import pallas as pl


def kernel(stage, p1_action, p1_character, p2_action, p2_character, W_stage, W_p1_action, W_p1_character, W_p2_action, W_p2_character):
    raise NotImplementedError("write your pallas kernel here")


# speedup vs baseline: 6.3432x; 6.3432x over previous
import functools

import jax
import jax.numpy as jnp
from jax import lax
from jax.experimental import pallas as pl
from jax.experimental.pallas import tpu as pltpu
from jax.experimental.pallas import tpu_sc as plsc

_B, _L = 4096, 50
_N = _B * _L
_ED = 32
_NT = 5
_NC, _NS = 2, 16
_NW = _NC * _NS
_BPW = _N // _NW
_CH = 128
_CPW = _BPW // _CH


def _build_sc_kernel():
    mesh = plsc.VectorSubcoreMesh(core_axis_name="c", subcore_axis_name="s")

    scratch = (
        [pltpu.VMEM((_CH,), jnp.int32) for _ in range(_NT)]
        + [pltpu.VMEM((_CH, _ED), jnp.float32) for _ in range(_NT)]
        + [pltpu.SemaphoreType.DMA]
    )

    @functools.partial(
        pl.kernel,
        out_type=jax.ShapeDtypeStruct((_N, _NT * _ED), jnp.float32),
        mesh=mesh,
        scratch_types=scratch,
        compiler_params=pltpu.CompilerParams(use_tc_tiling_on_sc=False),
    )
    def k(i0, i1, i2, i3, i4, w0, w1, w2, w3, w4, out_hbm,
          v0, v1, v2, v3, v4, r0, r1, r2, r3, r4, sem):
        idx_hbm = (i0, i1, i2, i3, i4)
        tabs = (w0, w1, w2, w3, w4)
        idx_v = (v0, v1, v2, v3, v4)
        rows = (r0, r1, r2, r3, r4)
        wid = lax.axis_index("s") * _NC + lax.axis_index("c")
        base0 = wid * _BPW

        @pl.loop(0, _CPW)
        def _(c):
            base = base0 + c * _CH
            for t in range(_NT):
                pltpu.sync_copy(idx_hbm[t].at[pl.ds(base, _CH)], idx_v[t])
            gathers = [
                pltpu.async_copy(tabs[t].at[idx_v[t]], rows[t], sem)
                for t in range(_NT)
            ]
            for g in gathers:
                g.wait()
            for t in range(_NT):
                pltpu.sync_copy(
                    rows[t], out_hbm.at[pl.ds(base, _CH), pl.ds(t * _ED, _ED)])

    return k


_sc_gather = _build_sc_kernel()


def kernel(stage, p1_action, p1_character, p2_action, p2_character,
           W_stage, W_p1_action, W_p1_character, W_p2_action, W_p2_character):
    idxs = [x.reshape(_N) for x in
            (stage, p1_action, p1_character, p2_action, p2_character)]
    out = _sc_gather(*idxs, W_stage, W_p1_action, W_p1_character,
                     W_p2_action, W_p2_character)
    return out.reshape(_B, _L, _NT * _ED)


# speedup vs baseline: 8.0278x; 1.2656x over previous
import functools

import jax
import jax.numpy as jnp
from jax import lax
from jax.experimental import pallas as pl
from jax.experimental.pallas import tpu as pltpu
from jax.experimental.pallas import tpu_sc as plsc

_B, _L = 4096, 50
_N = _B * _L
_ED = 32
_NT = 5
_NC, _NS = 2, 16
_NW = _NC * _NS
_BPW = _N // _NW
_CH = 128
_CPW = _BPW // _CH
_PAIRS = _CPW // 2


def _build_sc_kernel():
    mesh = plsc.VectorSubcoreMesh(core_axis_name="c", subcore_axis_name="s")

    scratch = (
        [pltpu.VMEM((_CH,), jnp.int32) for _ in range(2 * _NT)]
        + [pltpu.VMEM((_CH, _ED), jnp.float32) for _ in range(2 * _NT)]
        + [pltpu.SemaphoreType.DMA for _ in range(6)]
    )

    @functools.partial(
        pl.kernel,
        out_type=jax.ShapeDtypeStruct((_N, _NT * _ED), jnp.float32),
        mesh=mesh,
        scratch_types=scratch,
        compiler_params=pltpu.CompilerParams(use_tc_tiling_on_sc=False),
    )
    def k(i0, i1, i2, i3, i4, w0, w1, w2, w3, w4, out_hbm,
          v00, v01, v02, v03, v04, v10, v11, v12, v13, v14,
          r00, r01, r02, r03, r04, r10, r11, r12, r13, r14,
          si0, si1, sg0, sg1, sw0, sw1):
        idx_hbm = (i0, i1, i2, i3, i4)
        tabs = (w0, w1, w2, w3, w4)
        iv = ((v00, v01, v02, v03, v04), (v10, v11, v12, v13, v14))
        rv = ((r00, r01, r02, r03, r04), (r10, r11, r12, r13, r14))
        si = (si0, si1)
        sg = (sg0, sg1)
        sw = (sw0, sw1)
        wid = lax.axis_index("s") * _NC + lax.axis_index("c")
        base0 = wid * _BPW

        def start_idx(p, c):
            for t in range(_NT):
                pltpu.make_async_copy(
                    idx_hbm[t].at[pl.ds(base0 + c * _CH, _CH)],
                    iv[p][t], si[p]).start()

        def wait_idx(p):
            for t in range(_NT):
                pltpu.make_async_copy(
                    idx_hbm[t].at[pl.ds(base0, _CH)], iv[p][t], si[p]).wait()

        def start_gather(p):
            for t in range(_NT):
                pltpu.make_async_copy(
                    tabs[t].at[iv[p][t]], rv[p][t], sg[p]).start()

        def wait_gather(p):
            for t in range(_NT):
                pltpu.make_async_copy(
                    tabs[t].at[iv[p][t]], rv[p][t], sg[p]).wait()

        def start_write(p, c):
            for t in range(_NT):
                pltpu.make_async_copy(
                    rv[p][t],
                    out_hbm.at[pl.ds(base0 + c * _CH, _CH),
                               pl.ds(t * _ED, _ED)],
                    sw[p]).start()

        def wait_write(p):
            for t in range(_NT):
                pltpu.make_async_copy(
                    rv[p][t],
                    out_hbm.at[pl.ds(base0, _CH), pl.ds(t * _ED, _ED)],
                    sw[p]).wait()

        start_idx(0, 0)
        start_idx(1, 1)
        wait_idx(0)
        start_gather(0)

        @pl.loop(0, _PAIRS)
        def _(i):
            c0 = 2 * i
            wait_idx(1)

            @pl.when(i >= 1)
            def _():
                wait_write(1)

            start_gather(1)
            wait_gather(0)
            start_write(0, c0)

            @pl.when(i <= _PAIRS - 2)
            def _():
                start_idx(0, c0 + 2)

            wait_write(0)

            @pl.when(i <= _PAIRS - 2)
            def _():
                wait_idx(0)
                start_gather(0)

            wait_gather(1)
            start_write(1, c0 + 1)

            @pl.when(i <= _PAIRS - 2)
            def _():
                start_idx(1, c0 + 3)

        wait_write(1)

    return k


_sc_gather = _build_sc_kernel()


def kernel(stage, p1_action, p1_character, p2_action, p2_character,
           W_stage, W_p1_action, W_p1_character, W_p2_action, W_p2_character):
    idxs = [x.reshape(_N) for x in
            (stage, p1_action, p1_character, p2_action, p2_character)]
    out = _sc_gather(*idxs, W_stage, W_p1_action, W_p1_character,
                     W_p2_action, W_p2_character)
    return out.reshape(_B, _L, _NT * _ED)
